# baseline (speedup 1.0000x reference)
import jax
import jax.numpy as jnp
from jax.experimental import pallas as pl
from jax.experimental.pallas import tpu as pltpu


def kernel(frames):
    B, C, T, H, W = frames.shape
    S = T // 4
    BC = B * C
    D = H * W
    G = 4
    idx = jnp.asarray([(t * (T - 1)) // (S - 1) for t in range(S)],
                      dtype=jnp.int32)
    src = frames.reshape(BC, T, 1, D)

    def body(s_ref, in_ref, out_ref):
        out_ref[...] = in_ref[...]

    grid_spec = pltpu.PrefetchScalarGridSpec(
        num_scalar_prefetch=1,
        grid=(BC // G, S),
        in_specs=[pl.BlockSpec((G, 1, 1, D), lambda bc, t, s: (bc, s[t], 0, 0))],
        out_specs=pl.BlockSpec((G, 1, 1, D), lambda bc, t, s: (bc, t, 0, 0)),
    )
    slow = pl.pallas_call(
        body,
        grid_spec=grid_spec,
        out_shape=jax.ShapeDtypeStruct((BC, S, 1, D), frames.dtype),
    )(idx, src).reshape(B, C, S, H, W)
    return (slow, frames)


# speedup vs baseline: 2.8318x; 2.8318x over previous
import jax
import jax.numpy as jnp
from jax.experimental import pallas as pl
from jax.experimental.pallas import tpu as pltpu


def kernel(frames):
    B, C, T, H, W = frames.shape
    S = T // 4
    BC = B * C
    D = H * W
    ROWS = BC * S
    NBUF = 8
    AHEAD = 4

    idx = [(t * (T - 1)) // (S - 1) for t in range(S)]
    srow = [(r // S) * T + idx[r % S] for r in range(ROWS)]

    src = frames.reshape(BC * T, D)

    def body(in_hbm, out_hbm, buf, sin, sout):
        def gather(j):
            return pltpu.make_async_copy(in_hbm.at[srow[j]],
                                         buf.at[j % NBUF], sin.at[j % NBUF])

        def scatter(j):
            return pltpu.make_async_copy(buf.at[j % NBUF],
                                         out_hbm.at[j], sout.at[j % NBUF])

        waited = set()
        for j in range(AHEAD):
            gather(j).start()
        for j in range(ROWS):
            gather(j).wait()
            scatter(j).start()
            k = j + AHEAD
            if k < ROWS:
                p = k - NBUF
                if p >= 0:
                    scatter(p).wait()
                    waited.add(p)
                gather(k).start()
        for j in range(ROWS):
            if j not in waited:
                scatter(j).wait()

    slow = pl.pallas_call(
        body,
        in_specs=[pl.BlockSpec(memory_space=pl.ANY)],
        out_specs=pl.BlockSpec(memory_space=pl.ANY),
        out_shape=jax.ShapeDtypeStruct((ROWS, D), frames.dtype),
        scratch_shapes=[
            pltpu.VMEM((NBUF, D), frames.dtype),
            pltpu.SemaphoreType.DMA((NBUF,)),
            pltpu.SemaphoreType.DMA((NBUF,)),
        ],
    )(src).reshape(B, C, S, H, W)
    return (slow, frames)
